# baseline (speedup 1.0000x reference)
import jax
import jax.numpy as jnp
from jax.experimental import pallas as pl
from jax.experimental.pallas import tpu as pltpu

_D_MODEL = 768
_D_FF = 1536
_E = 8
_S = 2048
_BT = 256
_NB = _S // _BT
_INVALID = 1 << 30


def _router_kernel(logits_ref, idx_ref, w8_ref, rank8_ref, cnt_ref, aux_ref):
    l = logits_ref[...]
    lanes = jax.lax.broadcasted_iota(jnp.int32, l.shape, 1)
    v1 = jnp.max(l, axis=1, keepdims=True)
    i1 = jnp.min(jnp.where(l == v1, lanes, _E), axis=1, keepdims=True)
    l_wo1 = jnp.where(lanes == i1, -jnp.inf, l)
    v2 = jnp.max(l_wo1, axis=1, keepdims=True)
    i2 = jnp.min(jnp.where(l_wo1 == v2, lanes, _E), axis=1, keepdims=True)
    kept = jnp.where(l >= v2, l, -1000000000.0)
    e = jnp.exp(kept - v1)
    p = e / jnp.sum(e, axis=1, keepdims=True)
    sel = (lanes == i1) | (lanes == i2)
    idx_ref[...] = jnp.concatenate([i1, i2], axis=1)
    w8_ref[...] = jnp.where(sel, p, 0.0)
    r_io = jax.lax.broadcasted_iota(jnp.int32, (_S, _S), 0)
    c_io = jax.lax.broadcasted_iota(jnp.int32, (_S, _S), 1)
    tril = (c_io < r_io).astype(jnp.bfloat16)
    m8 = sel.astype(jnp.bfloat16)
    rank = jnp.dot(tril, m8, preferred_element_type=jnp.float32)
    rank8_ref[...] = jnp.where(sel, rank.astype(jnp.int32), _INVALID)
    cnt_ref[...] = jnp.sum(sel.astype(jnp.int32), axis=0, keepdims=True)
    usage = jnp.sum(p, axis=0, keepdims=True)
    imp = usage / jnp.sum(usage)
    mean = jnp.mean(imp)
    std = jnp.sqrt(jnp.mean((imp - mean) ** 2))
    aux_ref[...] = (std / (mean + 1e-10)).reshape(1, 1)


def _ffn_kernel(cnt_sref, xn_ref, w1_ref, b1_ref, w2_ref, b2_ref, w8_ref,
                rank8_ref, out_ref):
    e = pl.program_id(0)
    b = pl.program_id(1)

    @pl.when((e == 0) & (b == 0))
    def _():
        out_ref[...] = jnp.zeros_like(out_ref)

    @pl.when(b * _BT < cnt_sref[e])
    def _():
        lanes8 = jax.lax.broadcasted_iota(jnp.int32, (_S, _E), 1)
        sel_e = lanes8 == e
        rank_e = jnp.sum(jnp.where(sel_e, rank8_ref[...], 0), axis=1,
                         keepdims=True)
        w8col = jnp.sum(jnp.where(sel_e, w8_ref[...], 0.0), axis=1,
                        keepdims=True)
        jcol = jax.lax.broadcasted_iota(jnp.int32, (_S, _BT), 1) + b * _BT
        pt_mask = rank_e == jcol
        pt = pt_mask.astype(jnp.bfloat16)
        ptw = jnp.where(pt_mask, w8col, 0.0).astype(jnp.bfloat16)
        xb = jax.lax.dot_general(
            pt, xn_ref[...].astype(jnp.bfloat16),
            (((0,), (0,)), ((), ())),
            preferred_element_type=jnp.float32)
        h = jnp.dot(xb.astype(jnp.bfloat16), w1_ref[0].astype(jnp.bfloat16),
                    preferred_element_type=jnp.float32)
        h = h + b1_ref[0]
        h = h * jax.nn.sigmoid(h)
        y = jnp.dot(h.astype(jnp.bfloat16), w2_ref[0].astype(jnp.bfloat16),
                    preferred_element_type=jnp.float32)
        y = y + b2_ref[0]
        out_ref[...] += jnp.dot(ptw, y.astype(jnp.bfloat16),
                                preferred_element_type=jnp.float32)


def kernel(x, gate_w, gate_b, expert_w1, expert_b1, expert_w2, expert_b2):
    x = jnp.asarray(x, dtype=jnp.float32)
    scores = jnp.mean(x, axis=-1, keepdims=True)
    spiked_x = jnp.where(scores > 0.1, x, 0.0)
    xn = spiked_x / (jnp.sum(spiked_x, axis=-1, keepdims=True) + 1e-08)
    noise_key = jax.random.key(42)
    logits = jnp.einsum('bsd,de->bse', xn, gate_w) + gate_b
    logits = logits + jax.random.normal(
        jax.random.fold_in(noise_key, 1), logits.shape) * 0.01

    xn2 = xn.reshape(_S, _D_MODEL)
    logits2 = logits.reshape(_S, _E)

    idx, w8, rank8, cnt, aux = pl.pallas_call(
        _router_kernel,
        out_shape=(
            jax.ShapeDtypeStruct((_S, 2), jnp.int32),
            jax.ShapeDtypeStruct((_S, _E), jnp.float32),
            jax.ShapeDtypeStruct((_S, _E), jnp.int32),
            jax.ShapeDtypeStruct((1, _E), jnp.int32),
            jax.ShapeDtypeStruct((1, 1), jnp.float32),
        ),
    )(logits2)

    out = pl.pallas_call(
        _ffn_kernel,
        grid_spec=pltpu.PrefetchScalarGridSpec(
            num_scalar_prefetch=1,
            grid=(_E, _NB),
            in_specs=[
                pl.BlockSpec((_S, _D_MODEL), lambda e, b, *_: (0, 0)),
                pl.BlockSpec((1, _D_MODEL, _D_FF), lambda e, b, *_: (e, 0, 0)),
                pl.BlockSpec((1, 1, _D_FF), lambda e, b, *_: (e, 0, 0)),
                pl.BlockSpec((1, _D_FF, _D_MODEL), lambda e, b, *_: (e, 0, 0)),
                pl.BlockSpec((1, 1, _D_MODEL), lambda e, b, *_: (e, 0, 0)),
                pl.BlockSpec((_S, _E), lambda e, b, *_: (0, 0)),
                pl.BlockSpec((_S, _E), lambda e, b, *_: (0, 0)),
            ],
            out_specs=pl.BlockSpec((_S, _D_MODEL), lambda e, b, *_: (0, 0)),
        ),
        out_shape=jax.ShapeDtypeStruct((_S, _D_MODEL), jnp.float32),
    )(cnt.reshape(_E), xn2, expert_w1, expert_b1.reshape(_E, 1, _D_FF),
      expert_w2, expert_b2.reshape(_E, 1, _D_MODEL), w8, rank8)

    return (out.reshape(x.shape), idx.reshape(1, _S, 2), aux[0, 0])


# speedup vs baseline: 1.0911x; 1.0911x over previous
import jax
import jax.numpy as jnp
from jax.experimental import pallas as pl
from jax.experimental.pallas import tpu as pltpu

_D_MODEL = 768
_D_FF = 1536
_E = 8
_S = 2048
_BT = 256
_NB = _S // _BT
_INVALID = 1 << 30


def _router_kernel(logits_ref, idx_ref, w8_ref, rank8t_ref, w8t_ref, cnt_ref,
                   aux_ref):
    l = logits_ref[...]
    lanes = jax.lax.broadcasted_iota(jnp.int32, l.shape, 1)
    v1 = jnp.max(l, axis=1, keepdims=True)
    i1 = jnp.min(jnp.where(l == v1, lanes, _E), axis=1, keepdims=True)
    l_wo1 = jnp.where(lanes == i1, -jnp.inf, l)
    v2 = jnp.max(l_wo1, axis=1, keepdims=True)
    i2 = jnp.min(jnp.where(l_wo1 == v2, lanes, _E), axis=1, keepdims=True)
    kept = jnp.where(l >= v2, l, -1000000000.0)
    e = jnp.exp(kept - v1)
    p = e / jnp.sum(e, axis=1, keepdims=True)
    sel = (lanes == i1) | (lanes == i2)
    idx_ref[...] = jnp.concatenate([i1, i2], axis=1)
    w8_ref[...] = jnp.where(sel, p, 0.0)
    r_io = jax.lax.broadcasted_iota(jnp.int32, (_S, _S), 0)
    c_io = jax.lax.broadcasted_iota(jnp.int32, (_S, _S), 1)
    tril = (c_io < r_io).astype(jnp.bfloat16)
    m8 = sel.astype(jnp.bfloat16)
    rank = jnp.dot(tril, m8, preferred_element_type=jnp.float32)
    rank8 = jnp.where(sel, rank.astype(jnp.int32), _INVALID)
    rank8t_ref[...] = rank8.T
    w8t_ref[...] = jnp.where(sel, p, 0.0).T
    cnt_ref[...] = jnp.sum(sel.astype(jnp.int32), axis=0, keepdims=True)
    usage = jnp.sum(p, axis=0, keepdims=True)
    imp = usage / jnp.sum(usage)
    mean = jnp.mean(imp)
    std = jnp.sqrt(jnp.mean((imp - mean) ** 2))
    aux_ref[...] = (std / (mean + 1e-10)).reshape(1, 1)


def _ffn_kernel(cnt_sref, xn_ref, w1_ref, b1_ref, w2_ref, b2_ref, w8t_ref,
                rank8t_ref, out_ref):
    e = pl.program_id(0)
    b = pl.program_id(1)

    @pl.when((e == 0) & (b == 0))
    def _():
        out_ref[...] = jnp.zeros_like(out_ref)

    @pl.when(b * _BT < cnt_sref[e])
    def _():
        rank_row = rank8t_ref[pl.ds(e, 1), :]
        w_row = w8t_ref[pl.ds(e, 1), :]
        jsub = jax.lax.broadcasted_iota(jnp.int32, (_BT, _S), 0) + b * _BT
        pm = jsub == rank_row
        p = pm.astype(jnp.bfloat16)
        xb = jnp.dot(p, xn_ref[...].astype(jnp.bfloat16),
                     preferred_element_type=jnp.float32)
        h = jnp.dot(xb.astype(jnp.bfloat16), w1_ref[0].astype(jnp.bfloat16),
                    preferred_element_type=jnp.float32)
        h = h + b1_ref[0]
        h = h * jax.nn.sigmoid(h)
        y = jnp.dot(h.astype(jnp.bfloat16), w2_ref[0].astype(jnp.bfloat16),
                    preferred_element_type=jnp.float32)
        y = y + b2_ref[0]
        pw = (p * w_row).astype(jnp.bfloat16)
        out_ref[...] += jax.lax.dot_general(
            pw, y.astype(jnp.bfloat16), (((0,), (0,)), ((), ())),
            preferred_element_type=jnp.float32)


def kernel(x, gate_w, gate_b, expert_w1, expert_b1, expert_w2, expert_b2):
    x = jnp.asarray(x, dtype=jnp.float32)
    scores = jnp.mean(x, axis=-1, keepdims=True)
    spiked_x = jnp.where(scores > 0.1, x, 0.0)
    xn = spiked_x / (jnp.sum(spiked_x, axis=-1, keepdims=True) + 1e-08)
    noise_key = jax.random.key(42)
    logits = jnp.einsum('bsd,de->bse', xn, gate_w) + gate_b
    logits = logits + jax.random.normal(
        jax.random.fold_in(noise_key, 1), logits.shape) * 0.01

    xn2 = xn.reshape(_S, _D_MODEL)
    logits2 = logits.reshape(_S, _E)

    idx, w8, rank8t, w8t, cnt, aux = pl.pallas_call(
        _router_kernel,
        out_shape=(
            jax.ShapeDtypeStruct((_S, 2), jnp.int32),
            jax.ShapeDtypeStruct((_S, _E), jnp.float32),
            jax.ShapeDtypeStruct((_E, _S), jnp.int32),
            jax.ShapeDtypeStruct((_E, _S), jnp.float32),
            jax.ShapeDtypeStruct((1, _E), jnp.int32),
            jax.ShapeDtypeStruct((1, 1), jnp.float32),
        ),
    )(logits2)

    out = pl.pallas_call(
        _ffn_kernel,
        grid_spec=pltpu.PrefetchScalarGridSpec(
            num_scalar_prefetch=1,
            grid=(_E, _NB),
            in_specs=[
                pl.BlockSpec((_S, _D_MODEL), lambda e, b, *_: (0, 0)),
                pl.BlockSpec((1, _D_MODEL, _D_FF), lambda e, b, *_: (e, 0, 0)),
                pl.BlockSpec((1, 1, _D_FF), lambda e, b, *_: (e, 0, 0)),
                pl.BlockSpec((1, _D_FF, _D_MODEL), lambda e, b, *_: (e, 0, 0)),
                pl.BlockSpec((1, 1, _D_MODEL), lambda e, b, *_: (e, 0, 0)),
                pl.BlockSpec((_E, _S), lambda e, b, *_: (0, 0)),
                pl.BlockSpec((_E, _S), lambda e, b, *_: (0, 0)),
            ],
            out_specs=pl.BlockSpec((_S, _D_MODEL), lambda e, b, *_: (0, 0)),
        ),
        out_shape=jax.ShapeDtypeStruct((_S, _D_MODEL), jnp.float32),
    )(cnt.reshape(_E), xn2, expert_w1, expert_b1.reshape(_E, 1, _D_FF),
      expert_w2, expert_b2.reshape(_E, 1, _D_MODEL), w8t, rank8t)

    return (out.reshape(x.shape), idx.reshape(1, _S, 2), aux[0, 0])


# speedup vs baseline: 1.1877x; 1.0885x over previous
import jax
import jax.numpy as jnp
from jax.experimental import pallas as pl
from jax.experimental.pallas import tpu as pltpu

_D_MODEL = 768
_D_FF = 1536
_E = 8
_S = 2048


def _moe_kernel(logits_ref, xn_ref, w1_ref, b1_ref, w2_ref, b2_ref,
                out_ref, idx_ref, aux_ref, w8_scr):
    e = pl.program_id(0)

    @pl.when(e == 0)
    def _():
        l = logits_ref[...]
        lanes = jax.lax.broadcasted_iota(jnp.int32, l.shape, 1)
        v1 = jnp.max(l, axis=1, keepdims=True)
        i1 = jnp.min(jnp.where(l == v1, lanes, _E), axis=1, keepdims=True)
        l_wo1 = jnp.where(lanes == i1, -jnp.inf, l)
        v2 = jnp.max(l_wo1, axis=1, keepdims=True)
        i2 = jnp.min(jnp.where(l_wo1 == v2, lanes, _E), axis=1, keepdims=True)
        kept = jnp.where(l >= v2, l, -1000000000.0)
        ex = jnp.exp(kept - v1)
        p = ex / jnp.sum(ex, axis=1, keepdims=True)
        idx_ref[...] = jnp.concatenate([i1, i2], axis=1)
        w8_scr[...] = jnp.where((lanes == i1) | (lanes == i2), p, 0.0)
        usage = jnp.sum(p, axis=0, keepdims=True)
        imp = usage / jnp.sum(usage)
        mean = jnp.mean(imp)
        std = jnp.sqrt(jnp.mean((imp - mean) ** 2))
        aux_ref[...] = (std / (mean + 1e-10)).reshape(1, 1)

    sub8 = jax.lax.broadcasted_iota(jnp.int32, (_E, 1), 0)
    onehot = (sub8 == e).astype(jnp.float32)
    wsel = jnp.dot(w8_scr[...], onehot,
                   preferred_element_type=jnp.float32)
    h = jnp.dot(xn_ref[...].astype(jnp.bfloat16),
                w1_ref[0].astype(jnp.bfloat16),
                preferred_element_type=jnp.float32)
    h = h + b1_ref[0]
    h = h * jax.nn.sigmoid(h)
    y = jnp.dot(h.astype(jnp.bfloat16), w2_ref[0].astype(jnp.bfloat16),
                preferred_element_type=jnp.float32)
    y = y + b2_ref[0]

    @pl.when(e == 0)
    def _():
        out_ref[...] = wsel * y

    @pl.when(e > 0)
    def _():
        out_ref[...] += wsel * y


def kernel(x, gate_w, gate_b, expert_w1, expert_b1, expert_w2, expert_b2):
    x = jnp.asarray(x, dtype=jnp.float32)
    scores = jnp.mean(x, axis=-1, keepdims=True)
    spiked_x = jnp.where(scores > 0.1, x, 0.0)
    xn = spiked_x / (jnp.sum(spiked_x, axis=-1, keepdims=True) + 1e-08)
    noise_key = jax.random.key(42)
    logits = jnp.einsum('bsd,de->bse', xn, gate_w) + gate_b
    logits = logits + jax.random.normal(
        jax.random.fold_in(noise_key, 1), logits.shape) * 0.01

    xn2 = xn.reshape(_S, _D_MODEL)
    logits2 = logits.reshape(_S, _E)

    out, idx, aux = pl.pallas_call(
        _moe_kernel,
        grid=(_E,),
        in_specs=[
            pl.BlockSpec((_S, _E), lambda e: (0, 0)),
            pl.BlockSpec((_S, _D_MODEL), lambda e: (0, 0)),
            pl.BlockSpec((1, _D_MODEL, _D_FF), lambda e: (e, 0, 0)),
            pl.BlockSpec((1, 1, _D_FF), lambda e: (e, 0, 0)),
            pl.BlockSpec((1, _D_FF, _D_MODEL), lambda e: (e, 0, 0)),
            pl.BlockSpec((1, 1, _D_MODEL), lambda e: (e, 0, 0)),
        ],
        out_specs=(
            pl.BlockSpec((_S, _D_MODEL), lambda e: (0, 0)),
            pl.BlockSpec((_S, 2), lambda e: (0, 0)),
            pl.BlockSpec((1, 1), lambda e: (0, 0)),
        ),
        out_shape=(
            jax.ShapeDtypeStruct((_S, _D_MODEL), jnp.float32),
            jax.ShapeDtypeStruct((_S, 2), jnp.int32),
            jax.ShapeDtypeStruct((1, 1), jnp.float32),
        ),
        scratch_shapes=[pltpu.VMEM((_S, _E), jnp.float32)],
    )(logits2, xn2, expert_w1, expert_b1.reshape(_E, 1, _D_FF), expert_w2,
      expert_b2.reshape(_E, 1, _D_MODEL))

    return (out.reshape(x.shape), idx.reshape(1, _S, 2), aux[0, 0])


# speedup vs baseline: 1.1950x; 1.0061x over previous
import functools

import jax
import jax.numpy as jnp
from jax.experimental import pallas as pl
from jax.experimental.pallas import tpu as pltpu

_D_MODEL = 768
_D_FF = 1536
_E = 8
_S = 2048


def _router_kernel(logits_ref, idx_ref, w8_ref, aux_ref):
    l = logits_ref[...]
    lanes = jax.lax.broadcasted_iota(jnp.int32, l.shape, 1)
    v1 = jnp.max(l, axis=1, keepdims=True)
    i1 = jnp.min(jnp.where(l == v1, lanes, _E), axis=1, keepdims=True)
    l_wo1 = jnp.where(lanes == i1, -jnp.inf, l)
    v2 = jnp.max(l_wo1, axis=1, keepdims=True)
    i2 = jnp.min(jnp.where(l_wo1 == v2, lanes, _E), axis=1, keepdims=True)
    kept = jnp.where(l >= v2, l, -1000000000.0)
    e = jnp.exp(kept - v1)
    p = e / jnp.sum(e, axis=1, keepdims=True)
    idx_ref[...] = jnp.concatenate([i1, i2], axis=1)
    w8_ref[...] = jnp.where((lanes == i1) | (lanes == i2), p, 0.0)
    usage = jnp.sum(p, axis=0, keepdims=True)
    imp = usage / jnp.sum(usage)
    mean = jnp.mean(imp)
    std = jnp.sqrt(jnp.mean((imp - mean) ** 2))
    aux_ref[...] = (std / (mean + 1e-10)).reshape(1, 1)


def _ffn_kernel(xn_ref, w1_ref, b1_ref, w2_ref, b2_ref, w8_ref, out_ref):
    e = pl.program_id(0)

    @pl.when(e == 0)
    def _():
        out_ref[...] = jnp.zeros_like(out_ref)

    lanes = jax.lax.broadcasted_iota(jnp.int32, (_S, _E), 1)
    wsel = jnp.sum(jnp.where(lanes == e, w8_ref[...], 0.0), axis=1,
                   keepdims=True)
    h = jnp.dot(xn_ref[...], w1_ref[0].astype(jnp.bfloat16),
                preferred_element_type=jnp.float32)
    h = h + b1_ref[0]
    h = h * jax.nn.sigmoid(h)
    y = jnp.dot(h.astype(jnp.bfloat16), w2_ref[0].astype(jnp.bfloat16),
                preferred_element_type=jnp.float32)
    y = y + b2_ref[0]
    out_ref[...] += wsel * y


@functools.partial(jax.jit, static_argnums=())
def kernel(x, gate_w, gate_b, expert_w1, expert_b1, expert_w2, expert_b2):
    x = jnp.asarray(x, dtype=jnp.float32)
    scores = jnp.mean(x, axis=-1, keepdims=True)
    spiked_x = jnp.where(scores > 0.1, x, 0.0)
    xn = spiked_x / (jnp.sum(spiked_x, axis=-1, keepdims=True) + 1e-08)
    noise_key = jax.random.key(42)
    logits = jnp.einsum('bsd,de->bse', xn, gate_w) + gate_b
    logits = logits + jax.random.normal(
        jax.random.fold_in(noise_key, 1), logits.shape) * 0.01

    xn2 = xn.reshape(_S, _D_MODEL).astype(jnp.bfloat16)
    logits2 = logits.reshape(_S, _E)

    idx, w8, aux = pl.pallas_call(
        _router_kernel,
        out_shape=(
            jax.ShapeDtypeStruct((_S, 2), jnp.int32),
            jax.ShapeDtypeStruct((_S, _E), jnp.float32),
            jax.ShapeDtypeStruct((1, 1), jnp.float32),
        ),
    )(logits2)

    out = pl.pallas_call(
        _ffn_kernel,
        grid=(_E,),
        in_specs=[
            pl.BlockSpec((_S, _D_MODEL), lambda e: (0, 0)),
            pl.BlockSpec((1, _D_MODEL, _D_FF), lambda e: (e, 0, 0)),
            pl.BlockSpec((1, 1, _D_FF), lambda e: (e, 0, 0)),
            pl.BlockSpec((1, _D_FF, _D_MODEL), lambda e: (e, 0, 0)),
            pl.BlockSpec((1, 1, _D_MODEL), lambda e: (e, 0, 0)),
            pl.BlockSpec((_S, _E), lambda e: (0, 0)),
        ],
        out_specs=pl.BlockSpec((_S, _D_MODEL), lambda e: (0, 0)),
        out_shape=jax.ShapeDtypeStruct((_S, _D_MODEL), jnp.float32),
    )(xn2, expert_w1, expert_b1.reshape(_E, 1, _D_FF), expert_w2,
      expert_b2.reshape(_E, 1, _D_MODEL), w8)

    return (out.reshape(x.shape), idx.reshape(1, _S, 2), aux[0, 0])


# speedup vs baseline: 1.2503x; 1.0463x over previous
import jax
import jax.numpy as jnp
from jax.experimental import pallas as pl

_D_MODEL = 768
_D_FF = 1536
_E = 8
_S = 2048


def _router_kernel(x_ref, gw_ref, gb_ref, noise_ref, xnb_ref, idx_ref,
                   w8_ref, aux_ref):
    x = x_ref[...]
    scores = jnp.mean(x, axis=1, keepdims=True)
    spiked = jnp.where(scores > 0.1, x, 0.0)
    xn = spiked / (jnp.sum(spiked, axis=1, keepdims=True) + 1e-08)
    xnb_ref[...] = xn.astype(jnp.bfloat16)
    l = jnp.dot(xn, gw_ref[...], preferred_element_type=jnp.float32)
    l = l + gb_ref[...] + noise_ref[...]
    lanes = jax.lax.broadcasted_iota(jnp.int32, l.shape, 1)
    v1 = jnp.max(l, axis=1, keepdims=True)
    i1 = jnp.min(jnp.where(l == v1, lanes, _E), axis=1, keepdims=True)
    l_wo1 = jnp.where(lanes == i1, -jnp.inf, l)
    v2 = jnp.max(l_wo1, axis=1, keepdims=True)
    i2 = jnp.min(jnp.where(l_wo1 == v2, lanes, _E), axis=1, keepdims=True)
    kept = jnp.where(l >= v2, l, -1000000000.0)
    ex = jnp.exp(kept - v1)
    p = ex / jnp.sum(ex, axis=1, keepdims=True)
    idx_ref[...] = jnp.concatenate([i1, i2], axis=1)
    w8_ref[...] = jnp.where((lanes == i1) | (lanes == i2), p, 0.0)
    usage = jnp.sum(p, axis=0, keepdims=True)
    imp = usage / jnp.sum(usage)
    mean = jnp.mean(imp)
    std = jnp.sqrt(jnp.mean((imp - mean) ** 2))
    aux_ref[...] = (std / (mean + 1e-10)).reshape(1, 1)


def _ffn_kernel(xn_ref, w1_ref, b1_ref, w2_ref, b2_ref, w8_ref, out_ref):
    e = pl.program_id(0)

    @pl.when(e == 0)
    def _():
        out_ref[...] = jnp.zeros_like(out_ref)

    lanes = jax.lax.broadcasted_iota(jnp.int32, (_S, _E), 1)
    wsel = jnp.sum(jnp.where(lanes == e, w8_ref[...], 0.0), axis=1,
                   keepdims=True)
    h = jnp.dot(xn_ref[...], w1_ref[0].astype(jnp.bfloat16),
                preferred_element_type=jnp.float32)
    h = h + b1_ref[0]
    h = h * jax.nn.sigmoid(h)
    y = jnp.dot(h.astype(jnp.bfloat16), w2_ref[0].astype(jnp.bfloat16),
                preferred_element_type=jnp.float32)
    y = y + b2_ref[0]
    out_ref[...] += wsel * y


def kernel(x, gate_w, gate_b, expert_w1, expert_b1, expert_w2, expert_b2):
    x = jnp.asarray(x, dtype=jnp.float32)
    noise_key = jax.random.key(42)
    noise = jax.random.normal(
        jax.random.fold_in(noise_key, 1), (1, _S, _E)) * 0.01

    xnb, idx, w8, aux = pl.pallas_call(
        _router_kernel,
        out_shape=(
            jax.ShapeDtypeStruct((_S, _D_MODEL), jnp.bfloat16),
            jax.ShapeDtypeStruct((_S, 2), jnp.int32),
            jax.ShapeDtypeStruct((_S, _E), jnp.float32),
            jax.ShapeDtypeStruct((1, 1), jnp.float32),
        ),
    )(x.reshape(_S, _D_MODEL), gate_w, gate_b.reshape(1, _E),
      noise.reshape(_S, _E))

    out = pl.pallas_call(
        _ffn_kernel,
        grid=(_E,),
        in_specs=[
            pl.BlockSpec((_S, _D_MODEL), lambda e: (0, 0)),
            pl.BlockSpec((1, _D_MODEL, _D_FF), lambda e: (e, 0, 0)),
            pl.BlockSpec((1, 1, _D_FF), lambda e: (e, 0, 0)),
            pl.BlockSpec((1, _D_FF, _D_MODEL), lambda e: (e, 0, 0)),
            pl.BlockSpec((1, 1, _D_MODEL), lambda e: (e, 0, 0)),
            pl.BlockSpec((_S, _E), lambda e: (0, 0)),
        ],
        out_specs=pl.BlockSpec((_S, _D_MODEL), lambda e: (0, 0)),
        out_shape=jax.ShapeDtypeStruct((_S, _D_MODEL), jnp.float32),
    )(xnb, expert_w1, expert_b1.reshape(_E, 1, _D_FF), expert_w2,
      expert_b2.reshape(_E, 1, _D_MODEL), w8)

    return (out.reshape(x.shape), idx.reshape(1, _S, 2), aux[0, 0])
